# baseline (speedup 1.0000x reference)
import functools

import jax
import jax.numpy as jnp
from jax import lax
from jax.experimental import pallas as pl
from jax.experimental.pallas import tpu as pltpu
from jax.experimental.pallas import tpu_sc as plsc

N = 10000
D = 128
E = 320000
NUM_LAYERS = 4

NCORES = 2
NSUB = 16
NW = NCORES * NSUB
ROWS_PER_TILE = 632
NPAD = ROWS_PER_TILE * NSUB
CHUNK = 128
BLK = 16
NBLK = 5
NCHUNKS = BLK * NBLK
EPW = NCHUNKS * CHUNK
EPAD = NW * EPW
ZROWS = 40
NZ = 16


def _aggregate_sc(hh, src2d, dst2d):
    mesh = plsc.VectorSubcoreMesh(core_axis_name="c", subcore_axis_name="s")

    @functools.partial(
        pl.kernel,
        mesh=mesh,
        out_type=jax.ShapeDtypeStruct((NCORES * NPAD, D), jnp.float32),
        scratch_types=[
            pltpu.VMEM((BLK, CHUNK), jnp.int32),
            pltpu.VMEM((BLK, CHUNK), jnp.int32),
            pltpu.VMEM((BLK, CHUNK), jnp.int32),
            pltpu.VMEM((BLK, CHUNK), jnp.int32),
            pltpu.VMEM((CHUNK, D), jnp.float32),
            pltpu.VMEM((CHUNK, D), jnp.float32),
            pltpu.VMEM((ZROWS, D), jnp.float32),
            pltpu.VMEM_SHARED((NPAD, D), jnp.float32),
            pltpu.SemaphoreType.DMA,
            pltpu.SemaphoreType.DMA,
            pltpu.SemaphoreType.DMA,
            pltpu.SemaphoreType.DMA,
        ],
    )
    def k(hh_hbm, src_hbm, dst_hbm, out_hbm, sidx0, didx0, sidx1, didx1,
          rows_a, rows_b, zbuf, aggsh, sem_a, sem_b, sem_i, sem_z):
        sidx = (sidx0, sidx1)
        didx = (didx0, didx1)
        c = lax.axis_index("c")
        s = lax.axis_index("s")
        wid = c * NSUB + s
        row0 = s * ROWS_PER_TILE

        zv = jnp.zeros((16,), jnp.float32)
        for r in range(ZROWS):
            for q in range(D // 16):
                zbuf[r, pl.ds(q * 16, 16)] = zv
        for j in range(NZ - 1):
            pltpu.async_copy(zbuf, aggsh.at[pl.ds(row0 + j * ZROWS, ZROWS)],
                             sem_z)
        pltpu.async_copy(zbuf.at[pl.ds(0, 32)],
                         aggsh.at[pl.ds(row0 + (NZ - 1) * ZROWS, 32)], sem_z)

        def stage(blk):
            p = blk % 2
            pltpu.async_copy(
                src_hbm.at[pl.ds(wid * NCHUNKS + blk * BLK, BLK)], sidx[p],
                sem_i)
            pltpu.async_copy(
                dst_hbm.at[pl.ds(wid * NCHUNKS + blk * BLK, BLK)], didx[p],
                sem_i)

        def stage_wait(blk):
            p = blk % 2
            pltpu.make_async_copy(
                src_hbm.at[pl.ds(0, BLK)], sidx[p], sem_i).wait()
            pltpu.make_async_copy(
                src_hbm.at[pl.ds(0, BLK)], didx[p], sem_i).wait()

        def issue_g(buf, sem, blk, g):
            pltpu.async_copy(aggsh.at[sidx[blk % 2].at[g]], buf, sem)

        def wait_g(buf, sem):
            pltpu.make_async_copy(aggsh.at[sidx0.at[0]], buf, sem).wait()

        def scat(buf, blk, g):
            pltpu.sync_copy(buf, aggsh.at[didx[blk % 2].at[g]], add=True)

        stage(0)
        stage_wait(0)
        issue_g(rows_a, sem_a, 0, 0)
        issue_g(rows_b, sem_b, 0, 1)
        stage(1)
        for j in range(NZ):
            pltpu.make_async_copy(
                zbuf.at[pl.ds(0, 32)] if j == NZ - 1 else zbuf,
                aggsh.at[pl.ds(row0, 32 if j == NZ - 1 else ZROWS)],
                sem_z).wait()
        plsc.subcore_barrier()

        for blk in range(NBLK):
            def ebody(gp, carry, blk=blk):
                g = 2 + gp * 2
                wait_g(rows_a, sem_a)
                scat(rows_a, blk, g - 2)
                issue_g(rows_a, sem_a, blk, g)
                wait_g(rows_b, sem_b)
                scat(rows_b, blk, g - 1)
                issue_g(rows_b, sem_b, blk, g + 1)
                return carry

            lax.fori_loop(0, (BLK - 2) // 2, ebody, 0)
            wait_g(rows_a, sem_a)
            scat(rows_a, blk, BLK - 2)
            wait_g(rows_b, sem_b)
            scat(rows_b, blk, BLK - 1)
            if blk < NBLK - 1:
                stage_wait(blk + 1)
                issue_g(rows_a, sem_a, blk + 1, 0)
                issue_g(rows_b, sem_b, blk + 1, 1)
                if blk < NBLK - 2:
                    stage(blk + 2)

        plsc.subcore_barrier()
        pltpu.sync_copy(
            aggsh.at[pl.ds(row0, ROWS_PER_TILE)],
            out_hbm.at[pl.ds(c * NPAD + row0, ROWS_PER_TILE)],
        )

    return k(hh, src2d, dst2d)


def _mlp_body(hh_ref, agg_ref, w1_ref, g1_ref, b1_ref, w2_ref, go_ref, bo_ref,
              out_ref, pooled_ref, *p0_ref):
    agg = agg_ref[0:N, :] + agg_ref[NPAD:NPAD + N, :]
    z = hh_ref[...] + agg
    z = jnp.dot(z, w1_ref[...], preferred_element_type=jnp.float32,
                precision=lax.Precision.HIGHEST)
    mu = jnp.mean(z, axis=0, keepdims=True)
    var = jnp.mean((z - mu) ** 2, axis=0, keepdims=True)
    z = g1_ref[...] * (z - mu) * lax.rsqrt(var + 1e-5) + b1_ref[...]
    z = jnp.maximum(z, 0.0)
    z = jnp.dot(z, w2_ref[...], preferred_element_type=jnp.float32,
                precision=lax.Precision.HIGHEST)
    mu = jnp.mean(z, axis=0, keepdims=True)
    var = jnp.mean((z - mu) ** 2, axis=0, keepdims=True)
    z = go_ref[...] * (z - mu) * lax.rsqrt(var + 1e-5) + bo_ref[...]
    z = jnp.maximum(z, 0.0)
    out_ref[...] = z
    pooled_ref[...] = jnp.sum(z, axis=0, keepdims=True)
    if p0_ref:
        p0_ref[0][...] = jnp.sum(hh_ref[...], axis=0, keepdims=True)


def _mlp_tc(hh, aggflat, w1, g1, b1, w2, go, bo, first):
    out_shape = [
        jax.ShapeDtypeStruct((N, D), jnp.float32),
        jax.ShapeDtypeStruct((1, D), jnp.float32),
    ]
    if first:
        out_shape.append(jax.ShapeDtypeStruct((1, D), jnp.float32))
    return pl.pallas_call(
        _mlp_body,
        out_shape=out_shape,
    )(hh, aggflat, w1, g1.reshape(1, D), b1.reshape(1, D),
      w2, go.reshape(1, D), bo.reshape(1, D))


def _readout_body(p_ref, wp_ref, bp_ref, out_ref):
    p = p_ref[...]
    wp = wp_ref[...]
    acc = jnp.sum(bp_ref[...], axis=0, keepdims=True)
    for i in range(NUM_LAYERS + 1):
        acc = acc + jnp.dot(p[i:i + 1, :], wp[i], preferred_element_type=jnp.float32,
                            precision=lax.Precision.HIGHEST)
    out_ref[...] = acc


def _readout_tc(pooled_stack, wp_stack, bp_stack):
    return pl.pallas_call(
        _readout_body,
        out_shape=jax.ShapeDtypeStruct((1, D), jnp.float32),
    )(pooled_stack, wp_stack, bp_stack)


def kernel(h, edge_index, params):
    src = edge_index[0]
    dst = edge_index[1]
    npad_e = EPAD - E
    pad_iota = jnp.arange(npad_e, dtype=jnp.int32)
    src_p = jnp.concatenate([src, pad_iota % N])
    dst_p = jnp.concatenate([dst, N + pad_iota % (NPAD - N)])
    src2d = src_p.reshape(NW * NCHUNKS, CHUNK)
    dst2d = dst_p.reshape(NW * NCHUNKS, CHUNK)
    hh = h
    pooled = []
    for i in range(NUM_LAYERS):
        aggflat = _aggregate_sc(hh, src2d, dst2d)
        outs = _mlp_tc(hh, aggflat, params[f"W1_{i}"], params[f"g1_{i}"],
                       params[f"b1_{i}"], params[f"W2_{i}"], params[f"go_{i}"],
                       params[f"bo_{i}"], first=(i == 0))
        if i == 0:
            hh, p, p0 = outs
            pooled.append(p0)
        else:
            hh, p = outs
        pooled.append(p)
    pooled_stack = jnp.concatenate(pooled, axis=0)
    wp_stack = jnp.stack([params[f"Wp_{i}"] for i in range(NUM_LAYERS + 1)])
    bp_stack = jnp.stack([params[f"bp_{i}"] for i in range(NUM_LAYERS + 1)])
    return _readout_tc(pooled_stack, wp_stack, bp_stack)


# speedup vs baseline: 1.2845x; 1.2845x over previous
import functools

import jax
import jax.numpy as jnp
from jax import lax
from jax.experimental import pallas as pl
from jax.experimental.pallas import tpu as pltpu
from jax.experimental.pallas import tpu_sc as plsc

N = 10000
D = 128
E = 320000
NUM_LAYERS = 4

NCORES = 2
NSUB = 16
NW = NCORES * NSUB
ROWS_PER_TILE = 632
NPAD = ROWS_PER_TILE * NSUB
CHUNK = 128
BLK = 16
NBLK = 5
NCHUNKS = BLK * NBLK
EPW = NCHUNKS * CHUNK
EPAD = NW * EPW
ZROWS = 40
NZ = 16


def _aggregate_sc(hh, src2d, dst2d):
    mesh = plsc.VectorSubcoreMesh(core_axis_name="c", subcore_axis_name="s")

    @functools.partial(
        pl.kernel,
        mesh=mesh,
        out_type=jax.ShapeDtypeStruct((NCORES * NPAD, D), jnp.float32),
        scratch_types=[
            pltpu.VMEM((BLK, CHUNK), jnp.int32),
            pltpu.VMEM((BLK, CHUNK), jnp.int32),
            pltpu.VMEM((BLK, CHUNK), jnp.int32),
            pltpu.VMEM((BLK, CHUNK), jnp.int32),
            pltpu.VMEM((CHUNK, D), jnp.float32),
            pltpu.VMEM((CHUNK, D), jnp.float32),
            pltpu.VMEM((ZROWS, D), jnp.float32),
            pltpu.VMEM_SHARED((NPAD, D), jnp.float32),
            pltpu.SemaphoreType.DMA,
            pltpu.SemaphoreType.DMA,
            pltpu.SemaphoreType.DMA,
            pltpu.SemaphoreType.DMA,
        ],
    )
    def k(hh_hbm, src_hbm, dst_hbm, out_hbm, sidx0, didx0, sidx1, didx1,
          rows_a, rows_b, zbuf, aggsh, sem_a, sem_b, sem_i, sem_z):
        sidx = (sidx0, sidx1)
        didx = (didx0, didx1)
        c = lax.axis_index("c")
        s = lax.axis_index("s")
        wid = c * NSUB + s
        row0 = s * ROWS_PER_TILE

        zv = jnp.zeros((16,), jnp.float32)
        for r in range(ZROWS):
            for q in range(D // 16):
                zbuf[r, pl.ds(q * 16, 16)] = zv
        for j in range(NZ - 1):
            pltpu.async_copy(zbuf, aggsh.at[pl.ds(row0 + j * ZROWS, ZROWS)],
                             sem_z)
        pltpu.async_copy(zbuf.at[pl.ds(0, 32)],
                         aggsh.at[pl.ds(row0 + (NZ - 1) * ZROWS, 32)], sem_z)

        def stage(blk):
            p = blk % 2
            pltpu.async_copy(
                src_hbm.at[pl.ds(wid * NCHUNKS + blk * BLK, BLK)], sidx[p],
                sem_i)
            pltpu.async_copy(
                dst_hbm.at[pl.ds(wid * NCHUNKS + blk * BLK, BLK)], didx[p],
                sem_i)

        def stage_wait(blk):
            p = blk % 2
            pltpu.make_async_copy(
                src_hbm.at[pl.ds(0, BLK)], sidx[p], sem_i).wait()
            pltpu.make_async_copy(
                src_hbm.at[pl.ds(0, BLK)], didx[p], sem_i).wait()

        def issue_g(buf, sem, blk, g):
            pltpu.async_copy(hh_hbm.at[sidx[blk % 2].at[g]], buf, sem)

        def wait_g(buf, sem):
            pltpu.make_async_copy(hh_hbm.at[sidx0.at[0]], buf, sem).wait()

        def scat(buf, blk, g):
            pltpu.sync_copy(buf, aggsh.at[didx[blk % 2].at[g]], add=True)

        stage(0)
        stage_wait(0)
        issue_g(rows_a, sem_a, 0, 0)
        issue_g(rows_b, sem_b, 0, 1)
        stage(1)
        for j in range(NZ):
            pltpu.make_async_copy(
                zbuf.at[pl.ds(0, 32)] if j == NZ - 1 else zbuf,
                aggsh.at[pl.ds(row0, 32 if j == NZ - 1 else ZROWS)],
                sem_z).wait()
        plsc.subcore_barrier()

        for blk in range(NBLK):
            def ebody(gp, carry, blk=blk):
                g = 2 + gp * 2
                wait_g(rows_a, sem_a)
                scat(rows_a, blk, g - 2)
                issue_g(rows_a, sem_a, blk, g)
                wait_g(rows_b, sem_b)
                scat(rows_b, blk, g - 1)
                issue_g(rows_b, sem_b, blk, g + 1)
                return carry

            lax.fori_loop(0, (BLK - 2) // 2, ebody, 0)
            wait_g(rows_a, sem_a)
            scat(rows_a, blk, BLK - 2)
            wait_g(rows_b, sem_b)
            scat(rows_b, blk, BLK - 1)
            if blk < NBLK - 1:
                stage_wait(blk + 1)
                issue_g(rows_a, sem_a, blk + 1, 0)
                issue_g(rows_b, sem_b, blk + 1, 1)
                if blk < NBLK - 2:
                    stage(blk + 2)

        plsc.subcore_barrier()
        pltpu.sync_copy(
            aggsh.at[pl.ds(row0, ROWS_PER_TILE)],
            out_hbm.at[pl.ds(c * NPAD + row0, ROWS_PER_TILE)],
        )

    return k(hh, src2d, dst2d)


def _mlp_body(hh_ref, agg_ref, w1_ref, g1_ref, b1_ref, w2_ref, go_ref, bo_ref,
              out_ref, pooled_ref, *p0_ref):
    agg = agg_ref[0:N, :] + agg_ref[NPAD:NPAD + N, :]
    z = hh_ref[...] + agg
    z = jnp.dot(z, w1_ref[...], preferred_element_type=jnp.float32,
                precision=lax.Precision.HIGHEST)
    mu = jnp.mean(z, axis=0, keepdims=True)
    var = jnp.mean((z - mu) ** 2, axis=0, keepdims=True)
    z = g1_ref[...] * (z - mu) * lax.rsqrt(var + 1e-5) + b1_ref[...]
    z = jnp.maximum(z, 0.0)
    z = jnp.dot(z, w2_ref[...], preferred_element_type=jnp.float32,
                precision=lax.Precision.HIGHEST)
    mu = jnp.mean(z, axis=0, keepdims=True)
    var = jnp.mean((z - mu) ** 2, axis=0, keepdims=True)
    z = go_ref[...] * (z - mu) * lax.rsqrt(var + 1e-5) + bo_ref[...]
    z = jnp.maximum(z, 0.0)
    out_ref[...] = z
    pooled_ref[...] = jnp.sum(z, axis=0, keepdims=True)
    if p0_ref:
        p0_ref[0][...] = jnp.sum(hh_ref[...], axis=0, keepdims=True)


def _mlp_tc(hh, aggflat, w1, g1, b1, w2, go, bo, first):
    out_shape = [
        jax.ShapeDtypeStruct((N, D), jnp.float32),
        jax.ShapeDtypeStruct((1, D), jnp.float32),
    ]
    if first:
        out_shape.append(jax.ShapeDtypeStruct((1, D), jnp.float32))
    return pl.pallas_call(
        _mlp_body,
        out_shape=out_shape,
    )(hh, aggflat, w1, g1.reshape(1, D), b1.reshape(1, D),
      w2, go.reshape(1, D), bo.reshape(1, D))


def _readout_body(p_ref, wp_ref, bp_ref, out_ref):
    p = p_ref[...]
    wp = wp_ref[...]
    acc = jnp.sum(bp_ref[...], axis=0, keepdims=True)
    for i in range(NUM_LAYERS + 1):
        acc = acc + jnp.dot(p[i:i + 1, :], wp[i], preferred_element_type=jnp.float32,
                            precision=lax.Precision.HIGHEST)
    out_ref[...] = acc


def _readout_tc(pooled_stack, wp_stack, bp_stack):
    return pl.pallas_call(
        _readout_body,
        out_shape=jax.ShapeDtypeStruct((1, D), jnp.float32),
    )(pooled_stack, wp_stack, bp_stack)


def kernel(h, edge_index, params):
    src = edge_index[0]
    dst = edge_index[1]
    npad_e = EPAD - E
    pad_iota = jnp.arange(npad_e, dtype=jnp.int32)
    src_p = jnp.concatenate([src, pad_iota % N])
    dst_p = jnp.concatenate([dst, N + pad_iota % (NPAD - N)])
    src2d = src_p.reshape(NW * NCHUNKS, CHUNK)
    dst2d = dst_p.reshape(NW * NCHUNKS, CHUNK)
    hh = h
    pooled = []
    for i in range(NUM_LAYERS):
        aggflat = _aggregate_sc(hh, src2d, dst2d)
        outs = _mlp_tc(hh, aggflat, params[f"W1_{i}"], params[f"g1_{i}"],
                       params[f"b1_{i}"], params[f"W2_{i}"], params[f"go_{i}"],
                       params[f"bo_{i}"], first=(i == 0))
        if i == 0:
            hh, p, p0 = outs
            pooled.append(p0)
        else:
            hh, p = outs
        pooled.append(p)
    pooled_stack = jnp.concatenate(pooled, axis=0)
    wp_stack = jnp.stack([params[f"Wp_{i}"] for i in range(NUM_LAYERS + 1)])
    bp_stack = jnp.stack([params[f"bp_{i}"] for i in range(NUM_LAYERS + 1)])
    return _readout_tc(pooled_stack, wp_stack, bp_stack)


# speedup vs baseline: 1.2992x; 1.0114x over previous
import functools

import jax
import jax.numpy as jnp
from jax import lax
from jax.experimental import pallas as pl
from jax.experimental.pallas import tpu as pltpu
from jax.experimental.pallas import tpu_sc as plsc

N = 10000
D = 128
E = 320000
NUM_LAYERS = 4

NCORES = 2
NSUB = 16
NW = NCORES * NSUB
ROWS_PER_TILE = 632
NPAD = ROWS_PER_TILE * NSUB
CHUNK = 128
BLK = 16
NBLK = 5
NCHUNKS = BLK * NBLK
EPW = NCHUNKS * CHUNK
EPAD = NW * EPW
ZROWS = 40
NZ = 16


def _aggregate_sc(hh, src2d, dst2d):
    mesh = plsc.VectorSubcoreMesh(core_axis_name="c", subcore_axis_name="s")

    @functools.partial(
        pl.kernel,
        mesh=mesh,
        out_type=jax.ShapeDtypeStruct((NCORES * NPAD, D), jnp.float32),
        scratch_types=[
            pltpu.VMEM((BLK, CHUNK), jnp.int32),
            pltpu.VMEM((BLK, CHUNK), jnp.int32),
            pltpu.VMEM((BLK, CHUNK), jnp.int32),
            pltpu.VMEM((BLK, CHUNK), jnp.int32),
            pltpu.VMEM((CHUNK, D), jnp.float32),
            pltpu.VMEM((CHUNK, D), jnp.float32),
            pltpu.VMEM((ZROWS, D), jnp.float32),
            pltpu.VMEM_SHARED((NPAD, D), jnp.float32),
            pltpu.SemaphoreType.DMA,
            pltpu.SemaphoreType.DMA,
            pltpu.SemaphoreType.DMA,
            pltpu.SemaphoreType.DMA,
        ],
    )
    def k(hh_hbm, src_hbm, dst_hbm, out_hbm, sidx0, didx0, sidx1, didx1,
          rows_a, rows_b, zbuf, aggsh, sem_a, sem_b, sem_i, sem_z):
        sidx = (sidx0, sidx1)
        didx = (didx0, didx1)
        c = lax.axis_index("c")
        s = lax.axis_index("s")
        wid = c * NSUB + s
        row0 = s * ROWS_PER_TILE

        zv = jnp.zeros((16,), jnp.float32)
        for r in range(ZROWS):
            for q in range(D // 16):
                zbuf[r, pl.ds(q * 16, 16)] = zv
        for j in range(NZ - 1):
            pltpu.async_copy(zbuf, aggsh.at[pl.ds(row0 + j * ZROWS, ZROWS)],
                             sem_z)
        pltpu.async_copy(zbuf.at[pl.ds(0, 32)],
                         aggsh.at[pl.ds(row0 + (NZ - 1) * ZROWS, 32)], sem_z)

        def stage(blk):
            p = blk % 2
            pltpu.async_copy(
                src_hbm.at[pl.ds(wid * NCHUNKS + blk * BLK, BLK)], sidx[p],
                sem_i)
            pltpu.async_copy(
                dst_hbm.at[pl.ds(wid * NCHUNKS + blk * BLK, BLK)], didx[p],
                sem_i)

        def stage_wait(blk):
            p = blk % 2
            pltpu.make_async_copy(
                src_hbm.at[pl.ds(0, BLK)], sidx[p], sem_i).wait()
            pltpu.make_async_copy(
                src_hbm.at[pl.ds(0, BLK)], didx[p], sem_i).wait()

        def issue_g(buf, sem, blk, g):
            pltpu.async_copy(hh_hbm.at[sidx[blk % 2].at[g]], buf, sem)

        def wait_g(buf, sem):
            pltpu.make_async_copy(hh_hbm.at[sidx0.at[0]], buf, sem).wait()

        def scat(buf, blk, g):
            pltpu.sync_copy(buf, aggsh.at[didx[blk % 2].at[g]], add=True)

        stage(0)
        stage_wait(0)
        issue_g(rows_a, sem_a, 0, 0)
        issue_g(rows_b, sem_b, 0, 1)
        stage(1)
        for j in range(NZ):
            pltpu.make_async_copy(
                zbuf.at[pl.ds(0, 32)] if j == NZ - 1 else zbuf,
                aggsh.at[pl.ds(row0, 32 if j == NZ - 1 else ZROWS)],
                sem_z).wait()
        plsc.subcore_barrier()

        for blk in range(NBLK):
            def ebody(gp, carry, blk=blk):
                g = 2 + gp * 2
                wait_g(rows_a, sem_a)
                scat(rows_a, blk, g - 2)
                issue_g(rows_a, sem_a, blk, g)
                wait_g(rows_b, sem_b)
                scat(rows_b, blk, g - 1)
                issue_g(rows_b, sem_b, blk, g + 1)
                return carry

            lax.fori_loop(0, (BLK - 2) // 2, ebody, 0)
            wait_g(rows_a, sem_a)
            scat(rows_a, blk, BLK - 2)
            wait_g(rows_b, sem_b)
            scat(rows_b, blk, BLK - 1)
            if blk < NBLK - 1:
                stage_wait(blk + 1)
                issue_g(rows_a, sem_a, blk + 1, 0)
                issue_g(rows_b, sem_b, blk + 1, 1)
                if blk < NBLK - 2:
                    stage(blk + 2)

        plsc.subcore_barrier()
        pltpu.sync_copy(
            aggsh.at[pl.ds(row0, ROWS_PER_TILE)],
            out_hbm.at[pl.ds(c * NPAD + row0, ROWS_PER_TILE)],
        )

    return k(hh, src2d, dst2d)


def _mlp_body(hh_ref, agg_ref, w1_ref, g1_ref, b1_ref, w2_ref, go_ref, bo_ref,
              out_ref, pooled_ref, *p0_ref):
    agg = agg_ref[0:N, :] + agg_ref[NPAD:NPAD + N, :]
    z = hh_ref[...] + agg
    z = jnp.dot(z, w1_ref[...], preferred_element_type=jnp.float32,
                precision=lax.Precision.HIGHEST)
    mu = jnp.mean(z, axis=0, keepdims=True)
    var = jnp.mean((z - mu) ** 2, axis=0, keepdims=True)
    z = g1_ref[...] * (z - mu) * lax.rsqrt(var + 1e-5) + b1_ref[...]
    z = jnp.maximum(z, 0.0)
    z = jnp.dot(z, w2_ref[...], preferred_element_type=jnp.float32,
                precision=lax.Precision.HIGHEST)
    mu = jnp.mean(z, axis=0, keepdims=True)
    var = jnp.mean((z - mu) ** 2, axis=0, keepdims=True)
    z = go_ref[...] * (z - mu) * lax.rsqrt(var + 1e-5) + bo_ref[...]
    z = jnp.maximum(z, 0.0)
    out_ref[...] = z
    pooled_ref[...] = jnp.sum(z, axis=0, keepdims=True)
    if p0_ref:
        p0_ref[0][...] = jnp.sum(hh_ref[...], axis=0, keepdims=True)


def _mlp_tc(hh, aggflat, w1, g1, b1, w2, go, bo, first):
    out_shape = [
        jax.ShapeDtypeStruct((N, D), jnp.float32),
        jax.ShapeDtypeStruct((1, D), jnp.float32),
    ]
    if first:
        out_shape.append(jax.ShapeDtypeStruct((1, D), jnp.float32))
    return pl.pallas_call(
        _mlp_body,
        out_shape=out_shape,
    )(hh, aggflat, w1, g1.reshape(1, D), b1.reshape(1, D),
      w2, go.reshape(1, D), bo.reshape(1, D))


def _last_body(hh_ref, agg_ref, w1_ref, g1_ref, b1_ref, w2_ref, go_ref, bo_ref,
               p_ref, wp_ref, bp_ref, out_ref):
    agg = agg_ref[0:N, :] + agg_ref[NPAD:NPAD + N, :]
    z = hh_ref[...] + agg
    z = jnp.dot(z, w1_ref[...], preferred_element_type=jnp.float32,
                precision=lax.Precision.HIGHEST)
    mu = jnp.mean(z, axis=0, keepdims=True)
    var = jnp.mean((z - mu) ** 2, axis=0, keepdims=True)
    z = g1_ref[...] * (z - mu) * lax.rsqrt(var + 1e-5) + b1_ref[...]
    z = jnp.maximum(z, 0.0)
    z = jnp.dot(z, w2_ref[...], preferred_element_type=jnp.float32,
                precision=lax.Precision.HIGHEST)
    mu = jnp.mean(z, axis=0, keepdims=True)
    var = jnp.mean((z - mu) ** 2, axis=0, keepdims=True)
    z = go_ref[...] * (z - mu) * lax.rsqrt(var + 1e-5) + bo_ref[...]
    z = jnp.maximum(z, 0.0)
    p4 = jnp.sum(z, axis=0, keepdims=True)
    p = jnp.concatenate([p_ref[...], p4], axis=0)
    wp = wp_ref[...]
    acc = jnp.sum(bp_ref[...], axis=0, keepdims=True)
    for i in range(NUM_LAYERS + 1):
        acc = acc + jnp.dot(p[i:i + 1, :], wp[i],
                            preferred_element_type=jnp.float32,
                            precision=lax.Precision.HIGHEST)
    out_ref[...] = acc


def _last_tc(hh, aggflat, w1, g1, b1, w2, go, bo, pooled_stack, wp_stack,
             bp_stack):
    return pl.pallas_call(
        _last_body,
        out_shape=jax.ShapeDtypeStruct((1, D), jnp.float32),
    )(hh, aggflat, w1, g1.reshape(1, D), b1.reshape(1, D),
      w2, go.reshape(1, D), bo.reshape(1, D), pooled_stack, wp_stack,
      bp_stack)


def kernel(h, edge_index, params):
    src = edge_index[0]
    dst = edge_index[1]
    npad_e = EPAD - E
    pad_iota = jnp.arange(npad_e, dtype=jnp.int32)
    src_p = jnp.concatenate([src, pad_iota % N])
    dst_p = jnp.concatenate([dst, N + pad_iota % (NPAD - N)])
    src2d = src_p.reshape(NW * NCHUNKS, CHUNK)
    dst2d = dst_p.reshape(NW * NCHUNKS, CHUNK)
    hh = h
    pooled = []
    for i in range(NUM_LAYERS - 1):
        aggflat = _aggregate_sc(hh, src2d, dst2d)
        outs = _mlp_tc(hh, aggflat, params[f"W1_{i}"], params[f"g1_{i}"],
                       params[f"b1_{i}"], params[f"W2_{i}"], params[f"go_{i}"],
                       params[f"bo_{i}"], first=(i == 0))
        if i == 0:
            hh, p, p0 = outs
            pooled.append(p0)
        else:
            hh, p = outs
        pooled.append(p)
    i = NUM_LAYERS - 1
    aggflat = _aggregate_sc(hh, src2d, dst2d)
    pooled_stack = jnp.concatenate(pooled, axis=0)
    wp_stack = jnp.stack([params[f"Wp_{j}"] for j in range(NUM_LAYERS + 1)])
    bp_stack = jnp.stack([params[f"bp_{j}"] for j in range(NUM_LAYERS + 1)])
    return _last_tc(hh, aggflat, params[f"W1_{i}"], params[f"g1_{i}"],
                    params[f"b1_{i}"], params[f"W2_{i}"], params[f"go_{i}"],
                    params[f"bo_{i}"], pooled_stack, wp_stack, bp_stack)


# speedup vs baseline: 1.4472x; 1.1139x over previous
import functools

import jax
import jax.numpy as jnp
from jax import lax
from jax.experimental import pallas as pl
from jax.experimental.pallas import tpu as pltpu
from jax.experimental.pallas import tpu_sc as plsc

N = 10000
D = 128
E = 320000
NUM_LAYERS = 4

NCORES = 2
NSUB = 16
NW = NCORES * NSUB
ROWS_PER_TILE = 632
NPAD = ROWS_PER_TILE * NSUB
CHUNK = 128
BLK = 16
NBLK = 5
NCHUNKS = BLK * NBLK
EPW = NCHUNKS * CHUNK
EPAD = NW * EPW
ZROWS = 40
NZ = 16


def _aggregate_sc(hh, src2d, dst2d):
    mesh = plsc.VectorSubcoreMesh(core_axis_name="c", subcore_axis_name="s")

    @functools.partial(
        pl.kernel,
        mesh=mesh,
        out_type=jax.ShapeDtypeStruct((NCORES * NPAD, D), jnp.float32),
        scratch_types=[
            pltpu.VMEM((BLK, CHUNK), jnp.int32),
            pltpu.VMEM((BLK, CHUNK), jnp.int32),
            pltpu.VMEM((BLK, CHUNK), jnp.int32),
            pltpu.VMEM((BLK, CHUNK), jnp.int32),
            pltpu.VMEM((CHUNK, D), jnp.float32),
            pltpu.VMEM((CHUNK, D), jnp.float32),
            pltpu.VMEM((ZROWS, D), jnp.float32),
            pltpu.VMEM_SHARED((NPAD, D), jnp.float32),
            pltpu.SemaphoreType.DMA,
            pltpu.SemaphoreType.DMA,
            pltpu.SemaphoreType.DMA,
            pltpu.SemaphoreType.DMA,
        ],
    )
    def k(hh_hbm, src_hbm, dst_hbm, out_hbm, sidx0, didx0, sidx1, didx1,
          rows_a, rows_b, zbuf, aggsh, sem_a, sem_b, sem_i, sem_z):
        sidx = (sidx0, sidx1)
        didx = (didx0, didx1)
        c = lax.axis_index("c")
        s = lax.axis_index("s")
        wid = c * NSUB + s
        row0 = s * ROWS_PER_TILE

        zv = jnp.zeros((16,), jnp.float32)
        for r in range(ZROWS):
            for q in range(D // 16):
                zbuf[r, pl.ds(q * 16, 16)] = zv
        for j in range(NZ - 1):
            pltpu.async_copy(zbuf, aggsh.at[pl.ds(row0 + j * ZROWS, ZROWS)],
                             sem_z)
        pltpu.async_copy(zbuf.at[pl.ds(0, 32)],
                         aggsh.at[pl.ds(row0 + (NZ - 1) * ZROWS, 32)], sem_z)

        def stage(blk):
            p = blk % 2
            pltpu.async_copy(
                src_hbm.at[pl.ds(wid * NCHUNKS + blk * BLK, BLK)], sidx[p],
                sem_i)
            pltpu.async_copy(
                dst_hbm.at[pl.ds(wid * NCHUNKS + blk * BLK, BLK)], didx[p],
                sem_i)

        def stage_wait(blk):
            p = blk % 2
            pltpu.make_async_copy(
                src_hbm.at[pl.ds(0, BLK)], sidx[p], sem_i).wait()
            pltpu.make_async_copy(
                src_hbm.at[pl.ds(0, BLK)], didx[p], sem_i).wait()

        def issue_g(buf, sem, blk, g):
            pltpu.async_copy(hh_hbm.at[sidx[blk % 2].at[g]], buf, sem)

        def wait_g(buf, sem):
            pltpu.make_async_copy(hh_hbm.at[sidx0.at[0]], buf, sem).wait()

        def scat(buf, blk, g):
            pltpu.sync_copy(buf, aggsh.at[didx[blk % 2].at[g]], add=True)

        stage(0)
        stage_wait(0)
        issue_g(rows_a, sem_a, 0, 0)
        issue_g(rows_b, sem_b, 0, 1)
        stage(1)
        for j in range(NZ):
            pltpu.make_async_copy(
                zbuf.at[pl.ds(0, 32)] if j == NZ - 1 else zbuf,
                aggsh.at[pl.ds(row0, 32 if j == NZ - 1 else ZROWS)],
                sem_z).wait()
        plsc.subcore_barrier()

        for blk in range(NBLK):
            def ebody(gp, carry, blk=blk):
                g = 2 + gp * 2
                wait_g(rows_a, sem_a)
                scat(rows_a, blk, g - 2)
                issue_g(rows_a, sem_a, blk, g)
                wait_g(rows_b, sem_b)
                scat(rows_b, blk, g - 1)
                issue_g(rows_b, sem_b, blk, g + 1)
                return carry

            lax.fori_loop(0, (BLK - 2) // 2, ebody, 0)
            wait_g(rows_a, sem_a)
            scat(rows_a, blk, BLK - 2)
            wait_g(rows_b, sem_b)
            scat(rows_b, blk, BLK - 1)
            if blk < NBLK - 1:
                stage_wait(blk + 1)
                issue_g(rows_a, sem_a, blk + 1, 0)
                issue_g(rows_b, sem_b, blk + 1, 1)
                if blk < NBLK - 2:
                    stage(blk + 2)

        plsc.subcore_barrier()
        pltpu.sync_copy(
            aggsh.at[pl.ds(row0, ROWS_PER_TILE)],
            out_hbm.at[pl.ds(c * NPAD + row0, ROWS_PER_TILE)],
        )

    return k(hh, src2d, dst2d)


def _mlp_body(hh_ref, agg_ref, w1_ref, g1_ref, b1_ref, w2_ref, go_ref, bo_ref,
              out_ref, pooled_ref, *p0_ref):
    agg = agg_ref[0:N, :] + agg_ref[NPAD:NPAD + N, :]
    z = hh_ref[...] + agg
    z = jnp.dot(z, w1_ref[...], preferred_element_type=jnp.float32,
                precision=lax.Precision.DEFAULT)
    mu = jnp.mean(z, axis=0, keepdims=True)
    var = jnp.mean((z - mu) ** 2, axis=0, keepdims=True)
    z = g1_ref[...] * (z - mu) * lax.rsqrt(var + 1e-5) + b1_ref[...]
    z = jnp.maximum(z, 0.0)
    z = jnp.dot(z, w2_ref[...], preferred_element_type=jnp.float32,
                precision=lax.Precision.DEFAULT)
    mu = jnp.mean(z, axis=0, keepdims=True)
    var = jnp.mean((z - mu) ** 2, axis=0, keepdims=True)
    z = go_ref[...] * (z - mu) * lax.rsqrt(var + 1e-5) + bo_ref[...]
    z = jnp.maximum(z, 0.0)
    out_ref[...] = z
    pooled_ref[...] = jnp.sum(z, axis=0, keepdims=True)
    if p0_ref:
        p0_ref[0][...] = jnp.sum(hh_ref[...], axis=0, keepdims=True)


def _mlp_tc(hh, aggflat, w1, g1, b1, w2, go, bo, first):
    out_shape = [
        jax.ShapeDtypeStruct((N, D), jnp.float32),
        jax.ShapeDtypeStruct((1, D), jnp.float32),
    ]
    if first:
        out_shape.append(jax.ShapeDtypeStruct((1, D), jnp.float32))
    return pl.pallas_call(
        _mlp_body,
        out_shape=out_shape,
    )(hh, aggflat, w1, g1.reshape(1, D), b1.reshape(1, D),
      w2, go.reshape(1, D), bo.reshape(1, D))


def _last_body(hh_ref, agg_ref, w1_ref, g1_ref, b1_ref, w2_ref, go_ref, bo_ref,
               p_ref, wp_ref, bp_ref, out_ref):
    agg = agg_ref[0:N, :] + agg_ref[NPAD:NPAD + N, :]
    z = hh_ref[...] + agg
    z = jnp.dot(z, w1_ref[...], preferred_element_type=jnp.float32,
                precision=lax.Precision.DEFAULT)
    mu = jnp.mean(z, axis=0, keepdims=True)
    var = jnp.mean((z - mu) ** 2, axis=0, keepdims=True)
    z = g1_ref[...] * (z - mu) * lax.rsqrt(var + 1e-5) + b1_ref[...]
    z = jnp.maximum(z, 0.0)
    z = jnp.dot(z, w2_ref[...], preferred_element_type=jnp.float32,
                precision=lax.Precision.DEFAULT)
    mu = jnp.mean(z, axis=0, keepdims=True)
    var = jnp.mean((z - mu) ** 2, axis=0, keepdims=True)
    z = go_ref[...] * (z - mu) * lax.rsqrt(var + 1e-5) + bo_ref[...]
    z = jnp.maximum(z, 0.0)
    p4 = jnp.sum(z, axis=0, keepdims=True)
    p = jnp.concatenate([p_ref[...], p4], axis=0)
    wp = wp_ref[...]
    acc = jnp.sum(bp_ref[...], axis=0, keepdims=True)
    for i in range(NUM_LAYERS + 1):
        acc = acc + jnp.dot(p[i:i + 1, :], wp[i],
                            preferred_element_type=jnp.float32,
                            precision=lax.Precision.DEFAULT)
    out_ref[...] = acc


def _last_tc(hh, aggflat, w1, g1, b1, w2, go, bo, pooled_stack, wp_stack,
             bp_stack):
    return pl.pallas_call(
        _last_body,
        out_shape=jax.ShapeDtypeStruct((1, D), jnp.float32),
    )(hh, aggflat, w1, g1.reshape(1, D), b1.reshape(1, D),
      w2, go.reshape(1, D), bo.reshape(1, D), pooled_stack, wp_stack,
      bp_stack)


def kernel(h, edge_index, params):
    src = edge_index[0]
    dst = edge_index[1]
    npad_e = EPAD - E
    pad_iota = jnp.arange(npad_e, dtype=jnp.int32)
    src_p = jnp.concatenate([src, pad_iota % N])
    dst_p = jnp.concatenate([dst, N + pad_iota % (NPAD - N)])
    src2d = src_p.reshape(NW * NCHUNKS, CHUNK)
    dst2d = dst_p.reshape(NW * NCHUNKS, CHUNK)
    hh = h
    pooled = []
    for i in range(NUM_LAYERS - 1):
        aggflat = _aggregate_sc(hh, src2d, dst2d)
        outs = _mlp_tc(hh, aggflat, params[f"W1_{i}"], params[f"g1_{i}"],
                       params[f"b1_{i}"], params[f"W2_{i}"], params[f"go_{i}"],
                       params[f"bo_{i}"], first=(i == 0))
        if i == 0:
            hh, p, p0 = outs
            pooled.append(p0)
        else:
            hh, p = outs
        pooled.append(p)
    i = NUM_LAYERS - 1
    aggflat = _aggregate_sc(hh, src2d, dst2d)
    pooled_stack = jnp.concatenate(pooled, axis=0)
    wp_stack = jnp.stack([params[f"Wp_{j}"] for j in range(NUM_LAYERS + 1)])
    bp_stack = jnp.stack([params[f"bp_{j}"] for j in range(NUM_LAYERS + 1)])
    return _last_tc(hh, aggflat, params[f"W1_{i}"], params[f"g1_{i}"],
                    params[f"b1_{i}"], params[f"W2_{i}"], params[f"go_{i}"],
                    params[f"bo_{i}"], pooled_stack, wp_stack, bp_stack)
